# baseline (speedup 1.0000x reference)
import functools

import jax
import jax.numpy as jnp
from jax import lax
from jax.experimental import pallas as pl
from jax.experimental.pallas import tpu as pltpu
from jax.experimental.pallas import tpu_sc as plsc

_NQ = 8
_QF = 32
_EB = 128
_GB = 512
_NC = 2
_NS = 16
_N1 = 16384
_N2 = 4096
_H = 256


def _make_seg_sum(E, Ndst):
  NB = E // (_EB * _NS)
  NBG = E // (_GB * _NS)
  SB = _GB // _EB
  RZ = Ndst // _NS
  PP = _NQ // _NC
  mesh = plsc.VectorSubcoreMesh(core_axis_name="c", subcore_axis_name="s",
                                num_cores=_NC, num_subcores=_NS)

  @functools.partial(
      pl.kernel,
      out_type=[jax.ShapeDtypeStruct((_NQ, Ndst, _QF), jnp.float32),
                jax.ShapeDtypeStruct((_NC, Ndst, _QF), jnp.float32)],
      mesh=mesh,
      scratch_types=[
          pltpu.VMEM((NB, _EB), jnp.int32),
          pltpu.VMEM((NB, _EB), jnp.int32),
          pltpu.VMEM((NB, _EB), jnp.int32),
          pltpu.VMEM((_GB, _QF), jnp.float32),
          pltpu.VMEM((_GB, _QF), jnp.float32),
          pltpu.VMEM((_EB, _QF), jnp.float32),
          pltpu.VMEM((_EB, _QF), jnp.float32),
          pltpu.VMEM_SHARED((Ndst, _QF), jnp.float32),
          pltpu.SemaphoreType.DMA,
          pltpu.SemaphoreType.DMA,
      ],
      compiler_params=pltpu.CompilerParams(use_tc_tiling_on_sc=False))
  def seg(table, srcr, dstr, agg_out, deg_out,
          srcb, dstb, gidx, rowsA, rowsB, ones_v, z64, acc,
          semA, semB):
    c = lax.axis_index("c")
    s = lax.axis_index("s")

    def fillz(i, _):
      for j in range(_QF // 16):
        z64[i, pl.ds(j * 16, 16)] = jnp.zeros((16,), jnp.float32)
      return 0
    lax.fori_loop(0, _EB, fillz, 0)

    def fillo(i, _):
      for j in range(_QF // 16):
        ones_v[i, pl.ds(j * 16, 16)] = jnp.ones((16,), jnp.float32)
      return 0
    lax.fori_loop(0, _EB, fillo, 0)

    for r in range(RZ // _EB):
      pltpu.sync_copy(z64, acc.at[pl.ds(s * RZ + r * _EB, _EB)])

    pltpu.sync_copy(srcr.at[pl.ds(s * NB, NB)], srcb)
    pltpu.sync_copy(dstr.at[pl.ds(s * NB, NB)], dstb)
    plsc.subcore_barrier()

    def wait_gather(buf, sem):
      pltpu.make_async_copy(table.at[pl.ds(0, _GB)], buf, sem).wait()

    def fire_gather(g, buf, sem):
      for k in range(SB):
        pltpu.async_copy(table.at[gidx.at[g * SB + k]],
                         buf.at[pl.ds(k * _EB, _EB)], sem)

    def scatter(buf, g):
      for k in range(SB):
        pltpu.sync_copy(buf.at[pl.ds(k * _EB, _EB)],
                        acc.at[dstb.at[g * SB + k]], add=True)

    for p in range(PP):
      q = c * PP + p

      def mkidx(b, _):
        for j in range(_EB // 16):
          sl = pl.ds(j * 16, 16)
          gidx[b, sl] = srcb[b, sl] * _NQ + q
        return 0
      lax.fori_loop(0, NB, mkidx, 0)

      fire_gather(0, rowsA, semA)

      def body(i, _):
        g1 = 2 * i + 1
        g2 = jnp.minimum(2 * i + 2, NBG - 1)
        wait_gather(rowsA, semA)
        fire_gather(g1, rowsB, semB)
        scatter(rowsA, 2 * i)
        wait_gather(rowsB, semB)
        fire_gather(g2, rowsA, semA)
        scatter(rowsB, g1)
        return 0
      lax.fori_loop(0, NBG // 2, body, 0)
      wait_gather(rowsA, semA)

      plsc.subcore_barrier()
      for r in range(RZ // _EB):
        row0 = s * RZ + r * _EB
        pltpu.sync_copy(acc.at[pl.ds(row0, _EB)],
                        agg_out.at[q, pl.ds(row0, _EB)])
        pltpu.sync_copy(z64, acc.at[pl.ds(row0, _EB)])
      plsc.subcore_barrier()

    half = NB // _NC

    def dblock(b, _):
      pltpu.sync_copy(ones_v, acc.at[dstb.at[c * half + b]], add=True)
      return 0
    lax.fori_loop(0, half, dblock, 0)
    plsc.subcore_barrier()
    for r in range(RZ // _EB):
      row0 = s * RZ + r * _EB
      pltpu.sync_copy(acc.at[pl.ds(row0, _EB)],
                      deg_out.at[c, pl.ds(row0, _EB)])

  return seg


def _dense0(hd, agg, deg, ws, wn, b):
  BM = 512
  N = hd.shape[0]

  def body(hd_ref, agg_ref, deg_ref, ws_ref, wn_ref, b_ref, out_ref):
    neigh = jnp.concatenate([agg_ref[q] for q in range(_NQ)], axis=1)
    d = jnp.clip(deg_ref[0, :, 0:1] + deg_ref[1, :, 0:1], 1.0, None)
    neigh = neigh / d
    h = jnp.dot(hd_ref[...], ws_ref[...], preferred_element_type=jnp.float32)
    h = h + jnp.dot(neigh, wn_ref[...], preferred_element_type=jnp.float32)
    h = h + b_ref[...]
    out_ref[...] = jnp.maximum(h, 0.0)

  return pl.pallas_call(
      body,
      grid=(N // BM,),
      in_specs=[
          pl.BlockSpec((BM, _H), lambda i: (i, 0)),
          pl.BlockSpec((_NQ, BM, _QF), lambda i: (0, i, 0)),
          pl.BlockSpec((_NC, BM, _QF), lambda i: (0, i, 0)),
          pl.BlockSpec((_H, _H), lambda i: (0, 0)),
          pl.BlockSpec((_H, _H), lambda i: (0, 0)),
          pl.BlockSpec((1, _H), lambda i: (0, 0)),
      ],
      out_specs=pl.BlockSpec((BM, _H), lambda i: (i, 0)),
      out_shape=jax.ShapeDtypeStruct((N, _H), jnp.float32),
  )(hd, agg, deg, ws, wn, b)


def _head(hd, agg, deg, ws, wn, b, wp, wpp):
  BM = 512
  N = hd.shape[0]

  def body(hd_ref, agg_ref, deg_ref, ws_ref, wn_ref, b_ref, wp_ref, wpp_ref,
           out_ref):
    hd_v = hd_ref[...]
    neigh = jnp.concatenate([agg_ref[q] for q in range(_NQ)], axis=1)
    d = jnp.clip(deg_ref[0, :, 0:1] + deg_ref[1, :, 0:1], 1.0, None)
    neigh = neigh / d
    h = jnp.dot(hd_v, ws_ref[...], preferred_element_type=jnp.float32)
    h = h + jnp.dot(neigh, wn_ref[...], preferred_element_type=jnp.float32)
    h = h + b_ref[...]
    h2 = jnp.maximum(h, 0.0)
    nb = jnp.maximum(hd_v, 0.0)
    hcat = jnp.concatenate([h2, nb], axis=1)
    logits = lax.dot_general(hcat, wp_ref[...], (((1,), (1,)), ((), ())),
                             preferred_element_type=jnp.float32)
    m = jnp.max(logits, axis=1, keepdims=True)
    col = lax.broadcasted_iota(jnp.int32, logits.shape, 1)
    idx = jnp.min(jnp.where(logits == m, col, 8), axis=1, keepdims=True)
    allh = lax.dot_general(hcat, wpp_ref[...], (((1,), (1,)), ((), ())),
                           preferred_element_type=jnp.float32)
    acc = jnp.zeros((BM, 16), jnp.float32)
    for cc in range(8):
      sel = (idx == cc).astype(jnp.float32)
      acc = acc + sel * lax.slice(allh, (0, 16 * cc), (BM, 16 * cc + 16))
    out_ref[...] = acc

  return pl.pallas_call(
      body,
      grid=(N // BM,),
      in_specs=[
          pl.BlockSpec((BM, _H), lambda i: (i, 0)),
          pl.BlockSpec((_NQ, BM, _QF), lambda i: (0, i, 0)),
          pl.BlockSpec((_NC, BM, _QF), lambda i: (0, i, 0)),
          pl.BlockSpec((_H, _H), lambda i: (0, 0)),
          pl.BlockSpec((_H, _H), lambda i: (0, 0)),
          pl.BlockSpec((1, _H), lambda i: (0, 0)),
          pl.BlockSpec((8, 2 * _H), lambda i: (0, 0)),
          pl.BlockSpec((128, 2 * _H), lambda i: (0, 0)),
      ],
      out_specs=pl.BlockSpec((BM, 16), lambda i: (i, 0)),
      out_shape=jax.ShapeDtypeStruct((N, 16), jnp.float32),
  )(hd, agg, deg, ws, wn, b, wp, wpp)


def kernel(inputs, src0, dst0, src1, dst1, W_self0, W_neigh0, b0,
           W_self1, W_neigh1, b1, W_prompt, W_pp):
  N0 = inputs.shape[0]
  E0 = src0.shape[0]
  E1 = src1.shape[0]

  seg0 = _make_seg_sum(E0, _N1)
  agg0, deg0 = seg0(inputs.reshape(N0 * _NQ, _QF),
                    src0.reshape(-1, _EB), dst0.reshape(-1, _EB))
  h1 = _dense0(inputs[:_N1], agg0, deg0, W_self0, W_neigh0,
               b0.reshape(1, -1))

  seg1 = _make_seg_sum(E1, _N2)
  agg1, deg1 = seg1(h1.reshape(_N1 * _NQ, _QF),
                    src1.reshape(-1, _EB), dst1.reshape(-1, _EB))
  out = _head(h1[:_N2], agg1, deg1, W_self1, W_neigh1, b1.reshape(1, -1),
              W_prompt, W_pp.reshape(8 * 16, 2 * _H))
  return out


# speedup vs baseline: 1.0634x; 1.0634x over previous
import functools

import jax
import jax.numpy as jnp
from jax import lax
from jax.experimental import pallas as pl
from jax.experimental.pallas import tpu as pltpu
from jax.experimental.pallas import tpu_sc as plsc

_NQ = 8
_QF = 32
_EB = 128
_GB = 512
_NC = 2
_NS = 16
_N1 = 16384
_N2 = 4096
_H = 256


def _make_seg_sum(E, Ndst):
  NB = E // (_EB * _NS)
  NBG = E // (_GB * _NS)
  SB = _GB // _EB
  RZ = Ndst // _NS
  PP = _NQ // _NC
  mesh = plsc.VectorSubcoreMesh(core_axis_name="c", subcore_axis_name="s",
                                num_cores=_NC, num_subcores=_NS)

  @functools.partial(
      pl.kernel,
      out_type=[jax.ShapeDtypeStruct((_NQ, Ndst, _QF), jnp.float32),
                jax.ShapeDtypeStruct((_NC, Ndst, _QF), jnp.float32)],
      mesh=mesh,
      scratch_types=[
          pltpu.VMEM((NB, _EB), jnp.int32),
          pltpu.VMEM((NB, _EB), jnp.int32),
          pltpu.VMEM((NB, _EB), jnp.int32),
          pltpu.VMEM((_GB, _QF), jnp.float32),
          pltpu.VMEM((_GB, _QF), jnp.float32),
          pltpu.VMEM((_EB, _QF), jnp.float32),
          pltpu.VMEM((_EB, _QF), jnp.float32),
          pltpu.VMEM_SHARED((Ndst, _QF), jnp.float32),
          pltpu.SemaphoreType.DMA,
          pltpu.SemaphoreType.DMA,
      ],
      compiler_params=pltpu.CompilerParams(use_tc_tiling_on_sc=False))
  def seg(table, srcr, dstr, agg_out, deg_out,
          srcb, dstb, gidx, rowsA, rowsB, ones_v, z64, acc,
          semA, semB):
    c = lax.axis_index("c")
    s = lax.axis_index("s")

    def fillz(i, _):
      for j in range(_QF // 16):
        z64[i, pl.ds(j * 16, 16)] = jnp.zeros((16,), jnp.float32)
      return 0
    lax.fori_loop(0, _EB, fillz, 0)

    def fillo(i, _):
      for j in range(_QF // 16):
        ones_v[i, pl.ds(j * 16, 16)] = jnp.ones((16,), jnp.float32)
      return 0
    lax.fori_loop(0, _EB, fillo, 0)

    for r in range(RZ // _EB):
      pltpu.sync_copy(z64, acc.at[pl.ds(s * RZ + r * _EB, _EB)])

    pltpu.sync_copy(srcr.at[pl.ds(s * NB, NB)], srcb)
    pltpu.sync_copy(dstr.at[pl.ds(s * NB, NB)], dstb)
    plsc.subcore_barrier()

    def wait_gather(buf, sem):
      pltpu.make_async_copy(table.at[pl.ds(0, _GB)], buf, sem).wait()

    def fire_gather(g, buf, sem):
      for k in range(SB):
        pltpu.async_copy(table.at[gidx.at[g * SB + k]],
                         buf.at[pl.ds(k * _EB, _EB)], sem)

    def scatter(buf, g):
      for k in range(SB):
        pltpu.sync_copy(buf.at[pl.ds(k * _EB, _EB)],
                        acc.at[dstb.at[g * SB + k]], add=True)

    for p in range(PP):
      q = c * PP + p

      def mkidx(b, _):
        for j in range(_EB // 16):
          sl = pl.ds(j * 16, 16)
          gidx[b, sl] = srcb[b, sl] * _NQ + q
        return 0
      lax.fori_loop(0, NB, mkidx, 0)

      fire_gather(0, rowsA, semA)

      def body(i, _):
        g1 = 2 * i + 1
        g2 = jnp.minimum(2 * i + 2, NBG - 1)
        fire_gather(g1, rowsB, semB)
        wait_gather(rowsA, semA)
        scatter(rowsA, 2 * i)
        fire_gather(g2, rowsA, semA)
        wait_gather(rowsB, semB)
        scatter(rowsB, g1)
        return 0
      lax.fori_loop(0, NBG // 2, body, 0)
      wait_gather(rowsA, semA)

      plsc.subcore_barrier()
      for r in range(RZ // _EB):
        row0 = s * RZ + r * _EB
        pltpu.sync_copy(acc.at[pl.ds(row0, _EB)],
                        agg_out.at[q, pl.ds(row0, _EB)])
        pltpu.sync_copy(z64, acc.at[pl.ds(row0, _EB)])
      plsc.subcore_barrier()

    half = NB // _NC

    def dblock(b, _):
      pltpu.sync_copy(ones_v, acc.at[dstb.at[c * half + b]], add=True)
      return 0
    lax.fori_loop(0, half, dblock, 0)
    plsc.subcore_barrier()
    for r in range(RZ // _EB):
      row0 = s * RZ + r * _EB
      pltpu.sync_copy(acc.at[pl.ds(row0, _EB)],
                      deg_out.at[c, pl.ds(row0, _EB)])

  return seg


def _dense0(hd, agg, deg, ws, wn, b):
  BM = 512
  N = hd.shape[0]

  def body(hd_ref, agg_ref, deg_ref, ws_ref, wn_ref, b_ref, out_ref):
    neigh = jnp.concatenate([agg_ref[q] for q in range(_NQ)], axis=1)
    d = jnp.clip(deg_ref[0, :, 0:1] + deg_ref[1, :, 0:1], 1.0, None)
    neigh = neigh / d
    h = jnp.dot(hd_ref[...], ws_ref[...], preferred_element_type=jnp.float32)
    h = h + jnp.dot(neigh, wn_ref[...], preferred_element_type=jnp.float32)
    h = h + b_ref[...]
    out_ref[...] = jnp.maximum(h, 0.0)

  return pl.pallas_call(
      body,
      grid=(N // BM,),
      in_specs=[
          pl.BlockSpec((BM, _H), lambda i: (i, 0)),
          pl.BlockSpec((_NQ, BM, _QF), lambda i: (0, i, 0)),
          pl.BlockSpec((_NC, BM, _QF), lambda i: (0, i, 0)),
          pl.BlockSpec((_H, _H), lambda i: (0, 0)),
          pl.BlockSpec((_H, _H), lambda i: (0, 0)),
          pl.BlockSpec((1, _H), lambda i: (0, 0)),
      ],
      out_specs=pl.BlockSpec((BM, _H), lambda i: (i, 0)),
      out_shape=jax.ShapeDtypeStruct((N, _H), jnp.float32),
  )(hd, agg, deg, ws, wn, b)


def _head(hd, agg, deg, ws, wn, b, wp, wpp):
  BM = 512
  N = hd.shape[0]

  def body(hd_ref, agg_ref, deg_ref, ws_ref, wn_ref, b_ref, wp_ref, wpp_ref,
           out_ref):
    hd_v = hd_ref[...]
    neigh = jnp.concatenate([agg_ref[q] for q in range(_NQ)], axis=1)
    d = jnp.clip(deg_ref[0, :, 0:1] + deg_ref[1, :, 0:1], 1.0, None)
    neigh = neigh / d
    h = jnp.dot(hd_v, ws_ref[...], preferred_element_type=jnp.float32)
    h = h + jnp.dot(neigh, wn_ref[...], preferred_element_type=jnp.float32)
    h = h + b_ref[...]
    h2 = jnp.maximum(h, 0.0)
    nb = jnp.maximum(hd_v, 0.0)
    hcat = jnp.concatenate([h2, nb], axis=1)
    logits = lax.dot_general(hcat, wp_ref[...], (((1,), (1,)), ((), ())),
                             preferred_element_type=jnp.float32)
    m = jnp.max(logits, axis=1, keepdims=True)
    col = lax.broadcasted_iota(jnp.int32, logits.shape, 1)
    idx = jnp.min(jnp.where(logits == m, col, 8), axis=1, keepdims=True)
    allh = lax.dot_general(hcat, wpp_ref[...], (((1,), (1,)), ((), ())),
                           preferred_element_type=jnp.float32)
    acc = jnp.zeros((BM, 16), jnp.float32)
    for cc in range(8):
      sel = (idx == cc).astype(jnp.float32)
      acc = acc + sel * lax.slice(allh, (0, 16 * cc), (BM, 16 * cc + 16))
    out_ref[...] = acc

  return pl.pallas_call(
      body,
      grid=(N // BM,),
      in_specs=[
          pl.BlockSpec((BM, _H), lambda i: (i, 0)),
          pl.BlockSpec((_NQ, BM, _QF), lambda i: (0, i, 0)),
          pl.BlockSpec((_NC, BM, _QF), lambda i: (0, i, 0)),
          pl.BlockSpec((_H, _H), lambda i: (0, 0)),
          pl.BlockSpec((_H, _H), lambda i: (0, 0)),
          pl.BlockSpec((1, _H), lambda i: (0, 0)),
          pl.BlockSpec((8, 2 * _H), lambda i: (0, 0)),
          pl.BlockSpec((128, 2 * _H), lambda i: (0, 0)),
      ],
      out_specs=pl.BlockSpec((BM, 16), lambda i: (i, 0)),
      out_shape=jax.ShapeDtypeStruct((N, 16), jnp.float32),
  )(hd, agg, deg, ws, wn, b, wp, wpp)


def kernel(inputs, src0, dst0, src1, dst1, W_self0, W_neigh0, b0,
           W_self1, W_neigh1, b1, W_prompt, W_pp):
  N0 = inputs.shape[0]
  E0 = src0.shape[0]
  E1 = src1.shape[0]

  seg0 = _make_seg_sum(E0, _N1)
  agg0, deg0 = seg0(inputs.reshape(N0 * _NQ, _QF),
                    src0.reshape(-1, _EB), dst0.reshape(-1, _EB))
  h1 = _dense0(inputs[:_N1], agg0, deg0, W_self0, W_neigh0,
               b0.reshape(1, -1))

  seg1 = _make_seg_sum(E1, _N2)
  agg1, deg1 = seg1(h1.reshape(_N1 * _NQ, _QF),
                    src1.reshape(-1, _EB), dst1.reshape(-1, _EB))
  out = _head(h1[:_N2], agg1, deg1, W_self1, W_neigh1, b1.reshape(1, -1),
              W_prompt, W_pp.reshape(8 * 16, 2 * _H))
  return out
